# Initial kernel scaffold; baseline (speedup 1.0000x reference)
import jax
import jax.numpy as jnp
from jax.experimental import pallas as pl


def kernel(decoder_states, encoder_states, step):
    raise NotImplementedError("write your pallas kernel here")


# speedup vs baseline: 49.0419x; 49.0419x over previous
import jax
import jax.numpy as jnp
from jax.experimental import pallas as pl
from jax.experimental.pallas import tpu as pltpu


def _mask_body(val_ref, out_ref, *, rows_per_blk, enc_seqlen):
    i = pl.program_id(1)
    ncols = out_ref.shape[2]
    r = jax.lax.broadcasted_iota(jnp.int32, (1, rows_per_blk, ncols), 1)
    r = r + i * rows_per_blk
    c = jax.lax.broadcasted_iota(jnp.int32, (1, rows_per_blk, ncols), 2)
    tgt = jnp.where(r < enc_seqlen, r, 0)
    out_ref[...] = jnp.where(c == tgt, val_ref[0], jnp.float32(0.0))


def kernel(decoder_states, encoder_states, step):
    batch_size, enc_seqlen, _ = encoder_states.shape
    _, dec_seqlen, _ = decoder_states.shape
    val = (-jnp.asarray(step, jnp.int32)).astype(jnp.float32).reshape(1)

    rows_per_blk = 512
    nblk = dec_seqlen // rows_per_blk

    import functools
    body = functools.partial(
        _mask_body, rows_per_blk=rows_per_blk, enc_seqlen=enc_seqlen
    )
    return pl.pallas_call(
        body,
        grid=(batch_size, nblk),
        in_specs=[pl.BlockSpec(memory_space=pltpu.SMEM)],
        out_specs=pl.BlockSpec(
            (1, rows_per_blk, enc_seqlen), lambda b, i: (b, i, 0)
        ),
        out_shape=jax.ShapeDtypeStruct(
            (batch_size, dec_seqlen, enc_seqlen), jnp.float32
        ),
    )(val)
